# Initial kernel scaffold; baseline (speedup 1.0000x reference)
import jax
import jax.numpy as jnp
from jax.experimental import pallas as pl


def kernel(cls_scores, bbox_deltas, device):
    raise NotImplementedError("write your pallas kernel here")


# speedup vs baseline: 6.0527x; 6.0527x over previous
import functools

import jax
import jax.numpy as jnp
from jax import lax
from jax.experimental import pallas as pl
from jax.experimental.pallas import tpu as pltpu

_RATIOS = (0.5, 1.0, 2.0)
_SCALES = (8, 16, 32)
_IMAGE_SIZE = 1920
_NMS_PRE = 3000
_NMS_POST = 300
_THRESHOLD = 0.5
_NEG = -1e30
_LANES = 128


def _make_anchors(feat_stride, size):
    ratios = jnp.asarray(_RATIOS, jnp.float32)
    scales = jnp.asarray(_SCALES, jnp.float32)
    base = feat_stride * scales
    ws = (base[None, :] * jnp.sqrt(1.0 / ratios)[:, None]).reshape(-1)
    hs = (base[None, :] * jnp.sqrt(ratios)[:, None]).reshape(-1)
    ctr = (jnp.arange(size, dtype=jnp.float32) + 0.5) * feat_stride
    cy = ctr[:, None, None]
    cx = ctr[None, :, None]
    x1, y1, x2, y2 = jnp.broadcast_arrays(
        cx - ws / 2, cy - hs / 2, cx + ws / 2, cy + hs / 2)
    return jnp.stack([x1, y1, x2, y2], axis=-1)


def _scramble(arr, K, H, W):
    lead = arr.shape[:-4]
    a = arr.reshape(lead + (K, 4, H, W))
    perm = tuple(range(len(lead))) + tuple(
        len(lead) + p for p in (2, 3, 0, 1))
    return jnp.transpose(a, perm).reshape(lead + (H * W * K, 4))


def _nms_body(n_rows, n_post, k_top,
              s_ref, dx1, dy1, dx2, dy2, ax1, ay1, ax2, ay2,
              out_ref, rx1, ry1, rx2, ry2, areas, snms):
    hi = jnp.float32(_IMAGE_SIZE)
    rx1[...] = jnp.clip(dx1[0] + ax1[...], 0.0, hi)
    ry1[...] = jnp.clip(dy1[0] + ay1[...], 0.0, hi)
    rx2[...] = jnp.clip(dx2[0] + ax2[...], 0.0, hi)
    ry2[...] = jnp.clip(dy2[0] + ay2[...], 0.0, hi)
    areas[...] = (rx2[...] - rx1[...]) * (ry2[...] - ry1[...])

    s = s_ref[0]
    u = lax.bitcast_convert_type(s, jnp.int32)
    m = u ^ (jnp.right_shift(u, 31) & jnp.int32(0x7FFFFFFF))

    def count_ge(t):
        return jnp.sum((m >= t).astype(jnp.int32))

    p0 = jnp.where(count_ge(jnp.int32(0)) >= k_top,
                   jnp.int32(0), jnp.int32(-2**31))

    def bit_step(i, p):
        b = jnp.int32(30) - i
        t = p + jnp.left_shift(jnp.int32(1), b)
        return jnp.where(count_ge(t) >= k_top, t, p)

    t_key = lax.fori_loop(0, 31, bit_step, p0)

    c_gt = jnp.sum((m > t_key).astype(jnp.int32))
    need = jnp.int32(k_top) - c_gt
    eq = m == t_key
    fidx = (lax.broadcasted_iota(jnp.int32, m.shape, 0) * _LANES
            + lax.broadcasted_iota(jnp.int32, m.shape, 1))

    def idx_step(i, r):
        b = jnp.int32(17) - i
        t = r + jnp.left_shift(jnp.int32(1), b)
        cl = jnp.sum((eq & (fidx < t)).astype(jnp.int32))
        return jnp.where(cl < need, t, r)

    e = lax.fori_loop(0, 18, idx_step, jnp.int32(0))
    sel = (m > t_key) | (eq & (fidx <= e))
    snms[...] = jnp.where(sel, s, jnp.float32(_NEG))

    li = lax.broadcasted_iota(jnp.int32, (1, _LANES), 1)

    def nms_step(t, carry):
        sarr = snms[...]
        mval = jnp.max(sarr)
        fi = jnp.min(jnp.where(sarr == mval, fidx, jnp.int32(2**30)))
        valid = mval > jnp.float32(_NEG)
        row = fi // _LANES
        lane = fi % _LANES

        def pick(ref):
            rowv = ref[pl.ds(row, 1), :]
            return jnp.sum(jnp.where(li == lane, rowv, 0.0))

        bx1 = pick(rx1)
        by1 = pick(ry1)
        bx2 = pick(rx2)
        by2 = pick(ry2)
        ba = pick(areas)

        xx1 = jnp.maximum(rx1[...], bx1)
        yy1 = jnp.maximum(ry1[...], by1)
        xx2 = jnp.minimum(rx2[...], bx2)
        yy2 = jnp.minimum(ry2[...], by2)
        inter = (jnp.maximum(xx2 - xx1, 0.0) * jnp.maximum(yy2 - yy1, 0.0))
        iou = inter / (areas[...] + ba - inter + jnp.float32(1e-9))
        kill = (iou > jnp.float32(_THRESHOLD)) | (fidx == fi)
        snms[...] = jnp.where(kill, jnp.float32(_NEG), sarr)

        z = jnp.float32(0.0)
        kx1 = jnp.where(valid, bx1, z)
        ky1 = jnp.where(valid, by1, z)
        kx2 = jnp.where(valid, bx2, z)
        ky2 = jnp.where(valid, by2, z)
        ks = jnp.where(valid, mval, z)
        vals = jnp.where(
            li == 0, kx1,
            jnp.where(li == 1, ky1,
                      jnp.where(li == 2, kx2,
                                jnp.where(li == 3, ky2,
                                          jnp.where(li == 4, ks, z)))))
        out_ref[0, pl.ds(t, 1), :] = vals
        return carry

    lax.fori_loop(0, n_post, nms_step, jnp.int32(0))


def kernel(cls_scores, bbox_deltas, device):
    N, C, H, W = cls_scores.shape
    K = C
    A = C * H * W
    feat_stride = round(_IMAGE_SIZE / float(W))

    anchors = _make_anchors(feat_stride, W)
    anchors_flat = _scramble(anchors, K, H, W)

    deltas = jnp.transpose(bbox_deltas, (0, 2, 3, 1)).reshape(N, H, W, K, 4)
    deltas_flat = _scramble(deltas, K, H, W)

    scores_flat = cls_scores.reshape(N, A)

    n_rows = -(-A // _LANES)
    n_rows += (-n_rows) % 8
    tot = n_rows * _LANES
    pad = tot - A

    s_pad = jnp.concatenate(
        [scores_flat, jnp.full((N, pad), -jnp.inf, jnp.float32)],
        axis=1).reshape(N, n_rows, _LANES)

    def pad_plane(x):
        lead = x.shape[:-1]
        z = jnp.zeros(lead + (pad,), jnp.float32)
        return jnp.concatenate([x, z], axis=-1).reshape(
            lead + (n_rows, _LANES))

    d_planes = [pad_plane(deltas_flat[..., c]) for c in range(4)]
    a_planes = [pad_plane(anchors_flat[..., c]) for c in range(4)]

    out_rows = _NMS_POST + (-_NMS_POST) % 8

    img_spec = pl.BlockSpec((1, n_rows, _LANES), lambda i: (i, 0, 0))
    const_spec = pl.BlockSpec((n_rows, _LANES), lambda i: (0, 0))

    body = functools.partial(_nms_body, n_rows, _NMS_POST, _NMS_PRE)
    out = pl.pallas_call(
        body,
        grid=(N,),
        in_specs=[img_spec] + [img_spec] * 4 + [const_spec] * 4,
        out_specs=pl.BlockSpec((1, out_rows, _LANES), lambda i: (i, 0, 0)),
        out_shape=jax.ShapeDtypeStruct((N, out_rows, _LANES), jnp.float32),
        scratch_shapes=[pltpu.VMEM((n_rows, _LANES), jnp.float32)
                        for _ in range(6)],
        compiler_params=pltpu.CompilerParams(
            dimension_semantics=("parallel",)),
    )(s_pad, *d_planes, *a_planes)

    boxes = out[:, :_NMS_POST, 0:4]
    last_scores = out[N - 1, :_NMS_POST, 4]
    scores_col = jnp.broadcast_to(last_scores[None, :], (N, _NMS_POST))
    return jnp.concatenate([scores_col[..., None], boxes], axis=-1)
